# baseline (device time: 224070 ns/iter reference)
import jax
import jax.numpy as jnp
from jax import lax
from jax.experimental import pallas as pl
from jax.experimental.pallas import tpu as pltpu

N_DEV = 16
M = 4096
N = 2048
CHUNK = M // N_DEV
NSUB = 4
QTR = CHUNK // (2 * NSUB)


def kernel(x, w_mat, scale_x, scale_w):
    m, k_per = x.shape
    _, n = w_mat.shape

    def body(x_ref, w_ref, sx_ref, sw_ref, out_ref, *scratch):
        R = 2 * NSUB
        comms = scratch[0:R]
        sends = scratch[R:2 * R]
        owns = scratch[2 * R:3 * R]
        sems = scratch[3 * R:4 * R]
        credits = scratch[4 * R:5 * R]

        my = lax.axis_index("i")
        left = lax.rem(my + N_DEV - 1, N_DEV)
        right = lax.rem(my + 1, N_DEV)

        rings = []
        for q in range(NSUB):
            rings.append((q, q * QTR, right, left, +1))
            rings.append((NSUB + q, (NSUB + q) * QTR, left, right, -1))

        def rows(c, base):
            return pl.ds(c * CHUNK + base, QTR)

        def c_send(s, sign):
            return lax.rem(my - sign * s + 2 * N_DEV, N_DEV)

        def c_recv(s, sign):
            return lax.rem(my - sign * (s + 1) + 2 * N_DEV, N_DEV)

        def gemm_chunk(delta):
            c = lax.rem(my + delta + N_DEV, N_DEV)
            acc = jnp.dot(
                x_ref[pl.ds(c * CHUNK, CHUNK), :],
                w_ref[:, :],
                preferred_element_type=jnp.int32,
            )
            out_ref[pl.ds(c * CHUNK, CHUNK), :] = acc.astype(jnp.float32)

        def rs_rdma(i, slot, to):
            return pltpu.make_async_remote_copy(
                src_ref=sends[i].at[slot],
                dst_ref=comms[i].at[slot],
                send_sem=sems[i].at[0, slot],
                recv_sem=sems[i].at[1, slot],
                device_id=(to,),
                device_id_type=pl.DeviceIdType.MESH,
            )

        for d in (0, 1, -1):
            gemm_chunk(d)

        for i, base, _, _, _ in rings:
            sends[i][0] = out_ref[rows(my, base), :].astype(jnp.bfloat16)

        barrier_sem = pltpu.get_barrier_semaphore()
        pl.semaphore_signal(barrier_sem, inc=1, device_id=(left,),
                            device_id_type=pl.DeviceIdType.MESH)
        pl.semaphore_signal(barrier_sem, inc=1, device_id=(right,),
                            device_id_type=pl.DeviceIdType.MESH)
        pl.semaphore_wait(barrier_sem, 2)

        hist = {}
        for i, base, to, _, sign in rings:
            rd = rs_rdma(i, 0, to)
            rd.start()
            hist[(i, 0)] = rd
        for d in (2, -2, 3, -3, 4, -4, 5, -5, 6, -6, 7, -7, 8):
            gemm_chunk(d)

        for s in range(N_DEV - 1):
            slot = s % 4
            nslot = (s + 1) % 4
            for i, base, to, cto, sign in rings:
                hist[(i, s)].wait_recv()
                if s >= 3:
                    hist[(i, s - 3)].wait_send()
                cr = c_recv(s, sign)
                acc = (
                    out_ref[rows(cr, base), :]
                    + comms[i][slot].astype(jnp.float32)
                )
                out_ref[rows(cr, base), :] = acc
                if s < N_DEV - 2:
                    sends[i][nslot] = acc.astype(jnp.bfloat16)
                    if s >= 3:
                        pl.semaphore_wait(credits[i], 1)
                    rd = rs_rdma(i, nslot, to)
                    rd.start()
                    hist[(i, s + 1)] = rd
                pl.semaphore_signal(credits[i], inc=1, device_id=(cto,),
                                    device_id_type=pl.DeviceIdType.MESH)
        for i, *_ in rings:
            for u in (N_DEV - 4, N_DEV - 3, N_DEV - 2):
                hist[(i, u)].wait_send()
        for cr_sem in credits:
            pl.semaphore_wait(cr_sem, 4)

        sc = sx_ref[0] * sw_ref[0]
        for i, base, _, _, sign in rings:
            own = lax.rem(my + sign + N_DEV, N_DEV)
            y = out_ref[rows(own, base), :] * sc
            v = y * (1.0 / (1.0 + jnp.exp(-y)))
            out_ref[rows(own, base), :] = v
            owns[i][:, :] = v.astype(jnp.bfloat16)

        def ag_rdma(i, src, slot, to):
            return pltpu.make_async_remote_copy(
                src_ref=src,
                dst_ref=comms[i].at[slot],
                send_sem=sems[i].at[0, slot],
                recv_sem=sems[i].at[1, slot],
                device_id=(to,),
                device_id_type=pl.DeviceIdType.MESH,
            )

        for i, base, to, _, sign in rings:
            rd = ag_rdma(i, owns[i], 0, to)
            rd.start()
            hist[(i, 0)] = rd
        for s in range(N_DEV - 1):
            slot = s % 4
            nslot = (s + 1) % 4
            for i, base, to, cto, sign in rings:
                hist[(i, s)].wait_recv()
                if s < N_DEV - 2:
                    if s >= 2:
                        pl.semaphore_wait(credits[i], 1)
                    rd = ag_rdma(i, comms[i].at[slot], nslot, to)
                    rd.start()
                    hist[(i, s + 1)] = rd
                cg = c_send(s, sign)
                out_ref[rows(cg, base), :] = (
                    comms[i][slot].astype(jnp.float32)
                )
                hist[(i, s)].wait_send()
                pl.semaphore_signal(credits[i], inc=1, device_id=(cto,),
                                    device_id_type=pl.DeviceIdType.MESH)
        for cr_sem in credits:
            pl.semaphore_wait(cr_sem, 3)

    return pl.pallas_call(
        body,
        out_shape=jax.ShapeDtypeStruct((m, n), jnp.float32),
        in_specs=[
            pl.BlockSpec(memory_space=pltpu.VMEM),
            pl.BlockSpec(memory_space=pltpu.VMEM),
            pl.BlockSpec(memory_space=pltpu.SMEM),
            pl.BlockSpec(memory_space=pltpu.SMEM),
        ],
        out_specs=pl.BlockSpec(memory_space=pltpu.VMEM),
        scratch_shapes=(
            [pltpu.VMEM((4, QTR, N), jnp.bfloat16)] * (2 * NSUB)
            + [pltpu.VMEM((4, QTR, N), jnp.bfloat16)] * (2 * NSUB)
            + [pltpu.VMEM((QTR, N), jnp.bfloat16)] * (2 * NSUB)
            + [pltpu.SemaphoreType.DMA((2, 4))] * (2 * NSUB)
            + [pltpu.SemaphoreType.REGULAR] * (2 * NSUB)
        ),
        compiler_params=pltpu.CompilerParams(
            collective_id=0,
            vmem_limit_bytes=60 * 1024 * 1024,
        ),
    )(x, w_mat, scale_x, scale_w)


# device time: 222927 ns/iter; 1.0051x vs baseline; 1.0051x over previous
import jax
import jax.numpy as jnp
from jax import lax
from jax.experimental import pallas as pl
from jax.experimental.pallas import tpu as pltpu

N_DEV = 16
M = 4096
N = 2048
CHUNK = M // N_DEV
QTR = CHUNK // 4


def kernel(x, w_mat, scale_x, scale_w):
    m, k_per = x.shape
    _, n = w_mat.shape

    def body(x_ref, w_ref, sx_ref, sw_ref, out_ref, *scratch):
        comms = scratch[0:4]
        sends = scratch[4:8]
        owns = scratch[8:12]
        sems = scratch[12:16]
        credits = scratch[16:20]

        my = lax.axis_index("i")
        left = lax.rem(my + N_DEV - 1, N_DEV)
        right = lax.rem(my + 1, N_DEV)

        rings = [
            (0, 0 * QTR, right, left, +1),
            (2, 2 * QTR, left, right, -1),
            (1, 1 * QTR, right, left, +1),
            (3, 3 * QTR, left, right, -1),
        ]

        def rows(c, base):
            return pl.ds(c * CHUNK + base, QTR)

        def c_send(s, sign):
            return lax.rem(my - sign * s + 2 * N_DEV, N_DEV)

        def c_recv(s, sign):
            return lax.rem(my - sign * (s + 1) + 2 * N_DEV, N_DEV)

        def gemm_chunk(delta):
            c = lax.rem(my + delta + N_DEV, N_DEV)
            acc = jnp.dot(
                x_ref[pl.ds(c * CHUNK, CHUNK), :],
                w_ref[:, :],
                preferred_element_type=jnp.int32,
            )
            out_ref[pl.ds(c * CHUNK, CHUNK), :] = acc.astype(jnp.float32)

        def rs_rdma(i, slot, to):
            return pltpu.make_async_remote_copy(
                src_ref=sends[i].at[slot],
                dst_ref=comms[i].at[slot],
                send_sem=sems[i].at[0, slot],
                recv_sem=sems[i].at[1, slot],
                device_id=(to,),
                device_id_type=pl.DeviceIdType.MESH,
            )

        for d in (0, 1, -1):
            gemm_chunk(d)

        for i, base, _, _, _ in rings:
            sends[i][0] = out_ref[rows(my, base), :].astype(jnp.bfloat16)

        barrier_sem = pltpu.get_barrier_semaphore()
        pl.semaphore_signal(barrier_sem, inc=1, device_id=(left,),
                            device_id_type=pl.DeviceIdType.MESH)
        pl.semaphore_signal(barrier_sem, inc=1, device_id=(right,),
                            device_id_type=pl.DeviceIdType.MESH)
        pl.semaphore_wait(barrier_sem, 2)

        hist = {}
        for i, base, to, _, sign in rings:
            rd = rs_rdma(i, 0, to)
            rd.start()
            hist[(i, 0)] = rd
        for d in (2, -2, 3, -3, 4, -4, 5, -5, 6, -6, 7, -7, 8):
            gemm_chunk(d)

        for s in range(N_DEV - 1):
            slot = s % 4
            nslot = (s + 1) % 4
            for i, base, to, cto, sign in rings:
                hist[(i, s)].wait_recv()
                if s >= 3:
                    hist[(i, s - 3)].wait_send()
                cr = c_recv(s, sign)
                acc = (
                    out_ref[rows(cr, base), :]
                    + comms[i][slot].astype(jnp.float32)
                )
                out_ref[rows(cr, base), :] = acc
                if s < N_DEV - 2:
                    sends[i][nslot] = acc.astype(jnp.bfloat16)
                    if s >= 3:
                        pl.semaphore_wait(credits[i], 1)
                    rd = rs_rdma(i, nslot, to)
                    rd.start()
                    hist[(i, s + 1)] = rd
                pl.semaphore_signal(credits[i], inc=1, device_id=(cto,),
                                    device_id_type=pl.DeviceIdType.MESH)
        for i, *_ in rings:
            for u in (N_DEV - 4, N_DEV - 3, N_DEV - 2):
                hist[(i, u)].wait_send()
        for cr_sem in credits:
            pl.semaphore_wait(cr_sem, 4)

        sc = sx_ref[0] * sw_ref[0]
        for i, base, _, _, sign in rings:
            own = lax.rem(my + sign + N_DEV, N_DEV)
            y = out_ref[rows(own, base), :] * sc
            v = y * (1.0 / (1.0 + jnp.exp(-y)))
            out_ref[rows(own, base), :] = v
            owns[i][:, :] = v.astype(jnp.bfloat16)

        def ag_rdma(i, src, slot, to):
            return pltpu.make_async_remote_copy(
                src_ref=src,
                dst_ref=comms[i].at[slot],
                send_sem=sems[i].at[0, slot],
                recv_sem=sems[i].at[1, slot],
                device_id=(to,),
                device_id_type=pl.DeviceIdType.MESH,
            )

        for i, base, to, _, sign in rings:
            rd = ag_rdma(i, owns[i], 0, to)
            rd.start()
            hist[(i, 0)] = rd
        for s in range(N_DEV - 1):
            slot = s % 4
            nslot = (s + 1) % 4
            for i, base, to, cto, sign in rings:
                hist[(i, s)].wait_recv()
                if s < N_DEV - 2:
                    if s >= 2:
                        pl.semaphore_wait(credits[i], 1)
                    rd = ag_rdma(i, comms[i].at[slot], nslot, to)
                    rd.start()
                    hist[(i, s + 1)] = rd
                cg = c_send(s, sign)
                out_ref[rows(cg, base), :] = (
                    comms[i][slot].astype(jnp.float32)
                )
                hist[(i, s)].wait_send()
                pl.semaphore_signal(credits[i], inc=1, device_id=(cto,),
                                    device_id_type=pl.DeviceIdType.MESH)
        for cr_sem in credits:
            pl.semaphore_wait(cr_sem, 3)

    return pl.pallas_call(
        body,
        out_shape=jax.ShapeDtypeStruct((m, n), jnp.float32),
        in_specs=[
            pl.BlockSpec(memory_space=pltpu.VMEM),
            pl.BlockSpec(memory_space=pltpu.VMEM),
            pl.BlockSpec(memory_space=pltpu.SMEM),
            pl.BlockSpec(memory_space=pltpu.SMEM),
        ],
        out_specs=pl.BlockSpec(memory_space=pltpu.VMEM),
        scratch_shapes=(
            [pltpu.VMEM((4, QTR, N), jnp.bfloat16)] * 4
            + [pltpu.VMEM((4, QTR, N), jnp.bfloat16)] * 4
            + [pltpu.VMEM((QTR, N), jnp.bfloat16)] * 4
            + [pltpu.SemaphoreType.DMA((2, 4))] * 4
            + [pltpu.SemaphoreType.REGULAR] * 4
        ),
        compiler_params=pltpu.CompilerParams(
            collective_id=0,
            vmem_limit_bytes=60 * 1024 * 1024,
        ),
    )(x, w_mat, scale_x, scale_w)
